# baseline (speedup 1.0000x reference)
import functools

import jax
import jax.numpy as jnp
from jax import lax
from jax.experimental import pallas as pl
from jax.experimental.pallas import tpu as pltpu

N = 50000
E = 800000
B = 256
H = 256
P = 480
NUM_LAYERS = 4

ROW_BLK = 1000
N_BLKS = N // ROW_BLK


def _mlp_stats_body(h_ref, agg_ref, w1_ref, b1_ref, w2_ref, b2_ref,
                    t2_ref, stats_ref):
    i = pl.program_id(0)

    @pl.when(i == 0)
    def _():
        stats_ref[...] = jnp.zeros_like(stats_ref)

    m = h_ref[...] + agg_ref[...]
    t1 = jnp.maximum(
        jnp.dot(m, w1_ref[...], preferred_element_type=jnp.float32)
        + b1_ref[...], 0.0)
    t2 = (jnp.dot(t1, w2_ref[...], preferred_element_type=jnp.float32)
          + b2_ref[...])
    t2_ref[...] = t2
    s = jnp.sum(t2, axis=0, keepdims=True)
    ss = jnp.sum(t2 * t2, axis=0, keepdims=True)
    stats_ref[...] += jnp.concatenate([s, ss], axis=0)


def _mlp_stats(h, agg, w1, b1, w2, b2):
    hin = h.shape[1]
    return pl.pallas_call(
        _mlp_stats_body,
        grid=(N_BLKS,),
        in_specs=[
            pl.BlockSpec((ROW_BLK, hin), lambda i: (i, 0)),
            pl.BlockSpec((ROW_BLK, hin), lambda i: (i, 0)),
            pl.BlockSpec((hin, H), lambda i: (0, 0)),
            pl.BlockSpec((1, H), lambda i: (0, 0)),
            pl.BlockSpec((H, H), lambda i: (0, 0)),
            pl.BlockSpec((1, H), lambda i: (0, 0)),
        ],
        out_specs=[
            pl.BlockSpec((ROW_BLK, H), lambda i: (i, 0)),
            pl.BlockSpec((2, H), lambda i: (0, 0)),
        ],
        out_shape=[
            jax.ShapeDtypeStruct((N, H), jnp.float32),
            jax.ShapeDtypeStruct((2, H), jnp.float32),
        ],
    )(h, agg, w1, b1.reshape(1, H), w2, b2.reshape(1, H))


def _bn_relu_body(residual, t2_ref, stats_ref, gb_ref, hprev_ref, out_ref):
    mean = stats_ref[0:1, :] / N
    var = stats_ref[1:2, :] / N - mean * mean
    inv = lax.rsqrt(var + 1e-5) * gb_ref[0:1, :]
    y = (t2_ref[...] - mean) * inv + gb_ref[1:2, :]
    y = jnp.maximum(y, 0.0)
    if residual:
        y = y + hprev_ref[...]
    out_ref[...] = y


def _bn_relu(t2, stats, g, b, hprev, residual):
    gb = jnp.concatenate([g.reshape(1, H), b.reshape(1, H)], axis=0)
    return pl.pallas_call(
        functools.partial(_bn_relu_body, residual),
        grid=(N_BLKS,),
        in_specs=[
            pl.BlockSpec((ROW_BLK, H), lambda i: (i, 0)),
            pl.BlockSpec((2, H), lambda i: (0, 0)),
            pl.BlockSpec((2, H), lambda i: (0, 0)),
            pl.BlockSpec((ROW_BLK, H), lambda i: (i, 0)),
        ],
        out_specs=pl.BlockSpec((ROW_BLK, H), lambda i: (i, 0)),
        out_shape=jax.ShapeDtypeStruct((N, H), jnp.float32),
    )(t2, stats, gb, hprev)


def _head_body(drug_ref, pe_ref, pw_ref, pb_ref, lngb_ref,
               w1_ref, b1_ref, w2_ref, b2_ref, w3_ref, b3_ref, out_ref):
    drug = drug_ref[...]
    pv = (jnp.dot(pe_ref[...], pw_ref[...], preferred_element_type=jnp.float32)
          + pb_ref[...])
    mu = jnp.mean(pv, axis=-1, keepdims=True)
    vv = jnp.mean(pv * pv, axis=-1, keepdims=True) - mu * mu
    pv = (pv - mu) * lax.rsqrt(vv + 1e-5) * lngb_ref[0:1, :] + lngb_ref[1:2, :]
    pv = jnp.maximum(pv, 0.0)
    cat = jnp.concatenate([drug, pv], axis=1)
    z = jnp.maximum(
        jnp.dot(cat, w1_ref[...], preferred_element_type=jnp.float32)
        + b1_ref[...], 0.0)
    z = jnp.maximum(
        jnp.dot(z, w2_ref[...], preferred_element_type=jnp.float32)
        + b2_ref[...], 0.0)
    out_ref[...] = (
        jnp.dot(z, w3_ref[...], preferred_element_type=jnp.float32)
        + b3_ref[...])


def _head(drug, protein_emb, params):
    lngb = jnp.concatenate(
        [params['ln_g'].reshape(1, H), params['ln_b'].reshape(1, H)], axis=0)
    return pl.pallas_call(
        _head_body,
        out_shape=jax.ShapeDtypeStruct((B, 1), jnp.float32),
    )(drug, protein_emb, params['proj_W'], params['proj_b'].reshape(1, H),
      lngb,
      params['pred_W1'], params['pred_b1'].reshape(1, 1024),
      params['pred_W2'], params['pred_b2'].reshape(1, 512),
      params['pred_W3'], params['pred_b3'].reshape(1, 1))


def kernel(x, edge_index, batch, protein_emb, params):
    src = edge_index[0]
    dst = edge_index[1]

    xp = jnp.pad(x, ((0, 0), (0, 128 - x.shape[1])))
    w1_0 = jnp.pad(params['gin0_W1'], ((0, 128 - x.shape[1]), (0, 0)))

    h = xp
    for i in range(NUM_LAYERS):
        agg = jax.ops.segment_sum(h[src], dst, num_segments=N)
        w1 = w1_0 if i == 0 else params[f'gin{i}_W1']
        t2, stats = _mlp_stats(h, agg, w1, params[f'gin{i}_b1'],
                               params[f'gin{i}_W2'], params[f'gin{i}_b2'])
        h = _bn_relu(t2, stats, params[f'bn{i}_g'], params[f'bn{i}_b'],
                     h, residual=(i > 0))

    drug = jax.ops.segment_sum(h, batch, num_segments=B)
    return _head(drug, protein_emb, params)


# speedup vs baseline: 2.0360x; 2.0360x over previous
import functools

import jax
import jax.numpy as jnp
from jax import lax
from jax.experimental import pallas as pl
from jax.experimental.pallas import tpu as pltpu
from jax.experimental.pallas import tpu_sc as plsc

N = 50000
E = 800000
B = 256
H = 256
P = 480
NUM_LAYERS = 4

NSUB = 16
NCORE = 2
ROW_BLK = 1000
N_BLKS = N // ROW_BLK
CW = 16

EDGE_K = 1024
E_PAD = 802816
E_SUB = E_PAD // NSUB
E_BATCHES = E_SUB // EDGE_K

N_PAD = 51200
ACC_ROWS = 50176
ZERO_ROWS = ACC_ROWS // NSUB
TRASH = N

POOL_SUB = N_PAD // NSUB
POOL_K = 640
POOL_BATCHES = POOL_SUB // POOL_K
POOL_ACC = 384
POOL_TRASH = B

_SC_MESH = plsc.VectorSubcoreMesh(core_axis_name="c", subcore_axis_name="s")


def _sc_agg_body(nchunks, h_ref, src_ref, dst_ref, zeros_ref, out_ref,
                 sidx, didx, rows, acc, gsem):
    c = lax.axis_index("c")
    s = lax.axis_index("s")
    cc = nchunks // NCORE
    for j in range(cc):
        p = c * cc + j
        pltpu.sync_copy(zeros_ref, acc.at[pl.ds(s * ZERO_ROWS, ZERO_ROWS)])
        plsc.subcore_barrier()

        def batch_body(b, _):
            base = s * E_SUB + b * EDGE_K
            pltpu.sync_copy(src_ref.at[pl.ds(base, EDGE_K)], sidx)
            pltpu.async_copy(h_ref.at[p].at[sidx], rows, gsem).wait()
            pltpu.sync_copy(dst_ref.at[pl.ds(base, EDGE_K)], didx)
            pltpu.sync_copy(rows, acc.at[didx], add=True)
            return 0

        lax.fori_loop(0, E_BATCHES, batch_body, 0)
        plsc.subcore_barrier()
        pltpu.sync_copy(
            acc.at[pl.ds(s * ZERO_ROWS, ZERO_ROWS)],
            out_ref.at[p].at[pl.ds(s * ZERO_ROWS, ZERO_ROWS)])
        plsc.subcore_barrier()


def _sc_agg(h_blk, src_pad, dst_pad, zeros_blk):
    nchunks = h_blk.shape[0]
    return pl.kernel(
        functools.partial(_sc_agg_body, nchunks),
        out_type=jax.ShapeDtypeStruct((nchunks, ACC_ROWS, CW), jnp.float32),
        mesh=_SC_MESH,
        compiler_params=pltpu.CompilerParams(use_tc_tiling_on_sc=False),
        scratch_types=[
            pltpu.VMEM((EDGE_K,), jnp.int32),
            pltpu.VMEM((EDGE_K,), jnp.int32),
            pltpu.VMEM((EDGE_K, CW), jnp.float32),
            pltpu.VMEM_SHARED((ACC_ROWS, CW), jnp.float32),
            pltpu.SemaphoreType.DMA,
        ],
    )(h_blk, src_pad, dst_pad, zeros_blk)


def _sc_pool_body(nchunks, h_ref, bidx_ref, zeros_ref, out_ref,
                  didx, rows, acc):
    c = lax.axis_index("c")
    s = lax.axis_index("s")
    cc = nchunks // NCORE
    zr = POOL_ACC // NSUB
    for j in range(cc):
        p = c * cc + j
        pltpu.sync_copy(zeros_ref, acc.at[pl.ds(s * zr, zr)])
        plsc.subcore_barrier()

        def batch_body(b, _):
            base = s * POOL_SUB + b * POOL_K
            pltpu.sync_copy(h_ref.at[p].at[pl.ds(base, POOL_K)], rows)
            pltpu.sync_copy(bidx_ref.at[pl.ds(base, POOL_K)], didx)
            pltpu.sync_copy(rows, acc.at[didx], add=True)
            return 0

        lax.fori_loop(0, POOL_BATCHES, batch_body, 0)
        plsc.subcore_barrier()
        wr = B // NSUB
        pltpu.sync_copy(acc.at[pl.ds(s * wr, wr)],
                        out_ref.at[p].at[pl.ds(s * wr, wr)])
        plsc.subcore_barrier()


def _sc_pool(h_blk, batch_pad, zeros_pool):
    nchunks = h_blk.shape[0]
    return pl.kernel(
        functools.partial(_sc_pool_body, nchunks),
        out_type=jax.ShapeDtypeStruct((nchunks, B, CW), jnp.float32),
        mesh=_SC_MESH,
        compiler_params=pltpu.CompilerParams(use_tc_tiling_on_sc=False),
        scratch_types=[
            pltpu.VMEM((POOL_K,), jnp.int32),
            pltpu.VMEM((POOL_K, CW), jnp.float32),
            pltpu.VMEM_SHARED((POOL_ACC, CW), jnp.float32),
        ],
    )(h_blk, batch_pad, zeros_pool)


def _mlp_stats_body(nchunks, h_ref, agg_ref, w1_ref, b1_ref, w2_ref, b2_ref,
                    t2_ref, stats_ref):
    i = pl.program_id(0)

    @pl.when(i == 0)
    def _():
        stats_ref[...] = jnp.zeros_like(stats_ref)

    m = jnp.concatenate(
        [h_ref[j] + agg_ref[j] for j in range(nchunks)], axis=1)
    t1 = jnp.maximum(
        jnp.dot(m, w1_ref[...], preferred_element_type=jnp.float32)
        + b1_ref[...], 0.0)
    t2 = (jnp.dot(t1, w2_ref[...], preferred_element_type=jnp.float32)
          + b2_ref[...])
    t2_ref[...] = t2
    s = jnp.sum(t2, axis=0, keepdims=True)
    ss = jnp.sum(t2 * t2, axis=0, keepdims=True)
    stats_ref[...] += jnp.concatenate([s, ss], axis=0)


def _mlp_stats(h_blk, agg_blk, w1, b1, w2, b2):
    nchunks = h_blk.shape[0]
    hin = nchunks * CW
    return pl.pallas_call(
        functools.partial(_mlp_stats_body, nchunks),
        grid=(N_BLKS,),
        in_specs=[
            pl.BlockSpec((nchunks, ROW_BLK, CW), lambda i: (0, i, 0)),
            pl.BlockSpec((nchunks, ROW_BLK, CW), lambda i: (0, i, 0)),
            pl.BlockSpec((hin, H), lambda i: (0, 0)),
            pl.BlockSpec((1, H), lambda i: (0, 0)),
            pl.BlockSpec((H, H), lambda i: (0, 0)),
            pl.BlockSpec((1, H), lambda i: (0, 0)),
        ],
        out_specs=[
            pl.BlockSpec((ROW_BLK, H), lambda i: (i, 0)),
            pl.BlockSpec((2, H), lambda i: (0, 0)),
        ],
        out_shape=[
            jax.ShapeDtypeStruct((N, H), jnp.float32),
            jax.ShapeDtypeStruct((2, H), jnp.float32),
        ],
    )(h_blk, agg_blk, w1, b1.reshape(1, H), w2, b2.reshape(1, H))


def _bn_relu_body(residual, hp_chunks, t2_ref, stats_ref, gb_ref, hprev_ref,
                  out_ref):
    mean = stats_ref[0:1, :] / N
    var = stats_ref[1:2, :] / N - mean * mean
    inv = lax.rsqrt(var + 1e-5) * gb_ref[0:1, :]
    y = (t2_ref[...] - mean) * inv + gb_ref[1:2, :]
    y = jnp.maximum(y, 0.0)
    if residual:
        y = y + jnp.concatenate([hprev_ref[j] for j in range(hp_chunks)],
                                axis=1)
    for j in range(H // CW):
        out_ref[j] = y[:, j * CW:(j + 1) * CW]


def _bn_relu(t2, stats, g, b, hprev_blk, residual):
    gb = jnp.concatenate([g.reshape(1, H), b.reshape(1, H)], axis=0)
    hp_chunks = hprev_blk.shape[0]
    return pl.pallas_call(
        functools.partial(_bn_relu_body, residual, hp_chunks),
        grid=(N_BLKS,),
        in_specs=[
            pl.BlockSpec((ROW_BLK, H), lambda i: (i, 0)),
            pl.BlockSpec((2, H), lambda i: (0, 0)),
            pl.BlockSpec((2, H), lambda i: (0, 0)),
            pl.BlockSpec((hp_chunks, ROW_BLK, CW), lambda i: (0, i, 0)),
        ],
        out_specs=pl.BlockSpec((H // CW, ROW_BLK, CW), lambda i: (0, i, 0)),
        out_shape=jax.ShapeDtypeStruct((H // CW, N_PAD, CW), jnp.float32),
    )(t2, stats, gb, hprev_blk)


def _head_body(drug_ref, pe_ref, pw_ref, pb_ref, lngb_ref,
               w1_ref, b1_ref, w2_ref, b2_ref, w3_ref, b3_ref, out_ref):
    drug = jnp.concatenate([drug_ref[j] for j in range(H // CW)], axis=1)
    pv = (jnp.dot(pe_ref[...], pw_ref[...], preferred_element_type=jnp.float32)
          + pb_ref[...])
    mu = jnp.mean(pv, axis=-1, keepdims=True)
    vv = jnp.mean(pv * pv, axis=-1, keepdims=True) - mu * mu
    pv = (pv - mu) * lax.rsqrt(vv + 1e-5) * lngb_ref[0:1, :] + lngb_ref[1:2, :]
    pv = jnp.maximum(pv, 0.0)
    cat = jnp.concatenate([drug, pv], axis=1)
    z = jnp.maximum(
        jnp.dot(cat, w1_ref[...], preferred_element_type=jnp.float32)
        + b1_ref[...], 0.0)
    z = jnp.maximum(
        jnp.dot(z, w2_ref[...], preferred_element_type=jnp.float32)
        + b2_ref[...], 0.0)
    out_ref[...] = (
        jnp.dot(z, w3_ref[...], preferred_element_type=jnp.float32)
        + b3_ref[...])


def _head(drug_blk, protein_emb, params):
    lngb = jnp.concatenate(
        [params['ln_g'].reshape(1, H), params['ln_b'].reshape(1, H)], axis=0)
    return pl.pallas_call(
        _head_body,
        out_shape=jax.ShapeDtypeStruct((B, 1), jnp.float32),
    )(drug_blk, protein_emb, params['proj_W'],
      params['proj_b'].reshape(1, H), lngb,
      params['pred_W1'], params['pred_b1'].reshape(1, 1024),
      params['pred_W2'], params['pred_b2'].reshape(1, 512),
      params['pred_W3'], params['pred_b3'].reshape(1, 1))


def kernel(x, edge_index, batch, protein_emb, params):
    src = edge_index[0]
    dst = edge_index[1]

    src_pad = jnp.concatenate(
        [src, jnp.zeros((E_PAD - E,), jnp.int32)])
    dst_pad = jnp.concatenate(
        [dst, jnp.full((E_PAD - E,), TRASH, jnp.int32)])
    batch_pad = jnp.concatenate(
        [batch, jnp.full((N_PAD - N,), POOL_TRASH, jnp.int32)])
    zeros_blk = jnp.zeros((ZERO_ROWS, CW), jnp.float32)
    zeros_pool = jnp.zeros((POOL_ACC // NSUB, CW), jnp.float32)

    f_in = x.shape[1]
    c0 = 128 // CW
    xb = jnp.pad(x, ((0, N_PAD - N), (0, 128 - f_in)))
    xb = xb.reshape(N_PAD, c0, CW).transpose(1, 0, 2)
    w1_0 = jnp.pad(params['gin0_W1'], ((0, 128 - f_in), (0, 0)))

    h_blk = xb
    for i in range(NUM_LAYERS):
        agg_blk = _sc_agg(h_blk, src_pad, dst_pad, zeros_blk)
        w1 = w1_0 if i == 0 else params[f'gin{i}_W1']
        t2, stats = _mlp_stats(h_blk, agg_blk, w1, params[f'gin{i}_b1'],
                               params[f'gin{i}_W2'], params[f'gin{i}_b2'])
        h_blk = _bn_relu(t2, stats, params[f'bn{i}_g'], params[f'bn{i}_b'],
                         h_blk, residual=(i > 0))

    drug_blk = _sc_pool(h_blk, batch_pad, zeros_pool)
    return _head(drug_blk, protein_emb, params)
